# baseline (speedup 1.0000x reference)
import functools

import jax
import jax.numpy as jnp
from jax import lax
from jax.experimental import pallas as pl
from jax.experimental.pallas import tpu as pltpu
from jax.experimental.pallas import tpu_sc as plsc

_TILES = 16
_B = 80
_GRP = 8


def kernel(node_feat, edge_feat, edge_index):
    n, d = node_feat.shape
    e = edge_feat.shape[0]
    h = d // 2

    epw = e // _TILES
    nchunks = epw // _B
    njobs = 2 * nchunks
    n_pad = ((n + _TILES * 8 - 1) // (_TILES * 8)) * (_TILES * 8)
    rows_pt = n_pad // _TILES
    rows_last = n - (_TILES - 1) * rows_pt

    mesh = plsc.VectorSubcoreMesh(core_axis_name="c", subcore_axis_name="s")

    @functools.partial(
        pl.kernel,
        out_type=jax.ShapeDtypeStruct((n, d), jnp.float32),
        mesh=mesh,
        scratch_types=[
            pltpu.VMEM((2, _GRP, _B), jnp.int32),
            pltpu.VMEM((2, _GRP, _B), jnp.int32),
            pltpu.VMEM((4, _B), jnp.int32),
            pltpu.VMEM((4, _B), jnp.int32),
            pltpu.VMEM((4, _B, h), jnp.float32),
            pltpu.VMEM_SHARED((n_pad, h), jnp.float32),
            pltpu.SemaphoreType.DMA,
        ]
        + [pltpu.SemaphoreType.DMA] * 4
        + [pltpu.SemaphoreType.DMA] * 4
        + [pltpu.SemaphoreType.DMA] * 2,
    )
    def k(node_hbm, edge_hbm, sidx_hbm, didx_hbm, out_hbm,
          sstage, dstage, gidx, didx, buf, acc, zsem,
          g0, g1, g2, g3, s0, s1, s2, s3, l0, l1):
        gsem = [g0, g1, g2, g3]
        ssem = [s0, s1, s2, s3]
        lsem = [l0, l1]

        c = lax.axis_index("c")
        s = lax.axis_index("s")
        base0 = s * epw
        arow = s * rows_pt
        colbase = pl.multiple_of(c * h, h)

        def issue_ld(grp, st):
            pltpu.async_copy(sidx_hbm.at[s, pl.ds(grp * _GRP, _GRP)],
                             sstage.at[st], lsem[st])
            pltpu.async_copy(didx_hbm.at[s, pl.ds(grp * _GRP, _GRP)],
                             dstage.at[st], lsem[st])

        def wait_ld(st):
            pltpu.make_async_copy(sidx_hbm.at[s, pl.ds(0, _GRP)],
                                  sstage.at[st], lsem[st]).wait()
            pltpu.make_async_copy(didx_hbm.at[s, pl.ds(0, _GRP)],
                                  dstage.at[st], lsem[st]).wait()

        def gen_node(b, k_tr):
            g_tr = k_tr // _GRP
            rem = k_tr - g_tr * _GRP
            if isinstance(k_tr, int):
                st = g_tr % 2
                for j in range(0, _B, 16):
                    didx[b, pl.ds(j, 16)] = dstage[st, rem, pl.ds(j, 16)]
                    gidx[b, pl.ds(j, 16)] = sstage[st, rem, pl.ds(j, 16)]
                return
            for st in range(2):
                @pl.when(g_tr % 2 == st)
                def _():
                    for j in range(0, _B, 16):
                        didx[b, pl.ds(j, 16)] = dstage[st, rem, pl.ds(j, 16)]
                        gidx[b, pl.ds(j, 16)] = sstage[st, rem, pl.ds(j, 16)]

        def gen_edge(b, k_tr):
            g_tr = k_tr // _GRP
            rem = k_tr - g_tr * _GRP
            if isinstance(k_tr, int):
                st = g_tr % 2
                for j in range(0, _B, 16):
                    didx[b, pl.ds(j, 16)] = dstage[st, rem, pl.ds(j, 16)]
                return
            for st in range(2):
                @pl.when(g_tr % 2 == st)
                def _():
                    for j in range(0, _B, 16):
                        didx[b, pl.ds(j, 16)] = dstage[st, rem, pl.ds(j, 16)]

        def issue_g(b, k_tr):
            if b % 2 == 0:
                pltpu.async_copy(node_hbm.at[gidx.at[b], pl.ds(colbase, h)],
                                 buf.at[b], gsem[b])
            else:
                pltpu.async_copy(
                    edge_hbm.at[pl.ds(base0 + k_tr * _B, _B), pl.ds(colbase, h)],
                    buf.at[b], gsem[b])

        def wait_g(b):
            if b % 2 == 0:
                pltpu.make_async_copy(node_hbm.at[gidx.at[b], pl.ds(colbase, h)],
                                      buf.at[b], gsem[b]).wait()
            else:
                pltpu.make_async_copy(
                    edge_hbm.at[pl.ds(base0, _B), pl.ds(colbase, h)],
                    buf.at[b], gsem[b]).wait()

        def issue_sc(b):
            pltpu.async_copy(buf.at[b], acc.at[didx.at[b]], ssem[b], add=True)

        def wait_sc(b):
            pltpu.make_async_copy(buf.at[b], acc.at[didx.at[b]], ssem[b]).wait()

        issue_ld(0, 0)

        @pl.loop(0, _B)
        def _(r):
            for j in range(0, h, 16):
                buf[0, r, pl.ds(j, 16)] = jnp.zeros((16,), jnp.float32)

        nfull = rows_pt // _B
        ztail = rows_pt - nfull * _B
        for m in range(nfull):
            pltpu.async_copy(buf.at[0], acc.at[pl.ds(arow + m * _B, _B)], zsem)
        if ztail:
            pltpu.async_copy(buf.at[0, pl.ds(0, ztail)],
                             acc.at[pl.ds(arow + nfull * _B, ztail)], zsem)
        for m in range(nfull):
            pltpu.make_async_copy(buf.at[0],
                                  acc.at[pl.ds(arow + m * _B, _B)], zsem).wait()
        if ztail:
            pltpu.make_async_copy(buf.at[0, pl.ds(0, ztail)],
                                  acc.at[pl.ds(arow + nfull * _B, ztail)],
                                  zsem).wait()
        wait_ld(0)
        issue_ld(1, 1)
        gen_node(0, 0)
        issue_g(0, 0)
        gen_edge(1, 0)
        issue_g(1, 0)
        gen_node(2, 1)
        issue_g(2, 1)
        gen_edge(3, 1)
        issue_g(3, 1)
        plsc.subcore_barrier()
        wait_g(0)
        issue_sc(0)
        wait_g(1)
        issue_sc(1)
        wait_g(2)
        issue_sc(2)

        @pl.loop(4, njobs - 2, step=4)
        def _(g0_tr):
            kbase = g0_tr // 2
            for b in range(4):
                k_tr = kbase + (b // 2)
                if b % 2 == 0:
                    wait_sc(b)
                    g_tr = k_tr // _GRP
                    at_start = (k_tr - g_tr * _GRP) == 0
                    for stw in range(2):
                        @pl.when(jnp.logical_and(at_start, g_tr % 2 == stw))
                        def _():
                            wait_ld(stw)

                        @pl.when(jnp.logical_and(
                            jnp.logical_and(at_start, g_tr % 2 == stw),
                            g_tr + 1 < (nchunks + _GRP - 1) // _GRP))
                        def _():
                            issue_ld(g_tr + 1, 1 - stw)
                    gen_node(b, k_tr)
                    issue_g(b, k_tr)
                else:
                    wait_sc(b)
                    gen_edge(b, k_tr)
                    issue_g(b, k_tr)
                bp = (b + 3) % 4
                wait_g(bp)
                issue_sc(bp)

        wait_sc(0)
        gen_node(0, nchunks - 1)
        issue_g(0, nchunks - 1)
        wait_g(3)
        issue_sc(3)
        wait_sc(1)
        gen_edge(1, nchunks - 1)
        issue_g(1, nchunks - 1)
        wait_g(0)
        issue_sc(0)
        wait_g(1)
        issue_sc(1)
        wait_sc(2)
        wait_sc(3)
        wait_sc(0)
        wait_sc(1)

        plsc.subcore_barrier()

        def writeout(sizes):
            off = 0
            for i, sz in enumerate(sizes):
                wb = i % 2
                if i >= 2:
                    psz = sizes[i - 2]
                    poff = sum(sizes[:i - 2])
                    pltpu.make_async_copy(
                        buf.at[wb, pl.ds(0, psz)],
                        out_hbm.at[pl.ds(arow + poff, psz), pl.ds(colbase, h)],
                        gsem[wb]).wait()
                pltpu.sync_copy(acc.at[pl.ds(arow + off, sz)],
                                buf.at[wb, pl.ds(0, sz)])
                pltpu.async_copy(
                    buf.at[wb, pl.ds(0, sz)],
                    out_hbm.at[pl.ds(arow + off, sz), pl.ds(colbase, h)],
                    gsem[wb])
                off += sz
            for i in range(max(0, len(sizes) - 2), len(sizes)):
                wb = i % 2
                sz = sizes[i]
                poff = sum(sizes[:i])
                pltpu.make_async_copy(
                    buf.at[wb, pl.ds(0, sz)],
                    out_hbm.at[pl.ds(arow + poff, sz), pl.ds(colbase, h)],
                    gsem[wb]).wait()

        @pl.when(s < _TILES - 1)
        def _():
            writeout([_B] * (rows_pt // _B) + [rows_pt % _B])

        @pl.when(s == _TILES - 1)
        def _():
            writeout([_B] * (rows_last // _B) + [rows_last % _B])

    cpt = epw // _B
    gpt = (cpt + _GRP - 1) * 0 + ((cpt + _GRP - 1) // _GRP) * _GRP
    ids = edge_index.reshape(2, _TILES, cpt, _B)
    ids = jnp.pad(ids, ((0, 0), (0, 0), (0, gpt - cpt), (0, 0)))
    return k(node_feat, edge_feat, ids[0], ids[1])


# speedup vs baseline: 1.0111x; 1.0111x over previous
import functools

import jax
import jax.numpy as jnp
from jax import lax
from jax.experimental import pallas as pl
from jax.experimental.pallas import tpu as pltpu
from jax.experimental.pallas import tpu_sc as plsc

_TILES = 16
_B = 80


def kernel(node_feat, edge_feat, edge_index):
    n, d = node_feat.shape
    e = edge_feat.shape[0]
    h = d // 2

    epw = e // _TILES
    nchunks = epw // _B
    njobs = 2 * nchunks
    n_pad = ((n + _TILES * 8 - 1) // (_TILES * 8)) * (_TILES * 8)
    rows_pt = n_pad // _TILES
    rows_last = n - (_TILES - 1) * rows_pt

    mesh = plsc.VectorSubcoreMesh(core_axis_name="c", subcore_axis_name="s")

    @functools.partial(
        pl.kernel,
        out_type=jax.ShapeDtypeStruct((n, d), jnp.float32),
        mesh=mesh,
        scratch_types=[
            pltpu.VMEM((2, _B), jnp.int32),
            pltpu.VMEM((2, _B), jnp.int32),
            pltpu.VMEM((4, _B), jnp.int32),
            pltpu.VMEM((4, _B), jnp.int32),
            pltpu.VMEM((4, _B, h), jnp.float32),
            pltpu.VMEM_SHARED((n_pad, h), jnp.float32),
            pltpu.SemaphoreType.DMA,
        ]
        + [pltpu.SemaphoreType.DMA] * 4
        + [pltpu.SemaphoreType.DMA] * 4
        + [pltpu.SemaphoreType.DMA] * 2,
    )
    def k(node_hbm, edge_hbm, eidx_hbm, out_hbm,
          sstage, dstage, gidx, didx, buf, acc, zsem,
          g0, g1, g2, g3, s0, s1, s2, s3, l0, l1):
        gsem = [g0, g1, g2, g3]
        ssem = [s0, s1, s2, s3]
        lsem = [l0, l1]

        c = lax.axis_index("c")
        s = lax.axis_index("s")
        base0 = s * epw
        arow = s * rows_pt
        colbase = pl.multiple_of(c * h, h)

        def issue_ld(k_tr, st):
            pltpu.async_copy(eidx_hbm.at[pl.ds(base0 + k_tr * _B, _B)],
                             sstage.at[st], lsem[st])
            pltpu.async_copy(eidx_hbm.at[pl.ds(e + base0 + k_tr * _B, _B)],
                             dstage.at[st], lsem[st])

        def wait_ld(st):
            pltpu.make_async_copy(eidx_hbm.at[pl.ds(base0, _B)],
                                  sstage.at[st], lsem[st]).wait()
            pltpu.make_async_copy(eidx_hbm.at[pl.ds(e + base0, _B)],
                                  dstage.at[st], lsem[st]).wait()

        def gen_node(b, st):
            for j in range(0, _B, 16):
                didx[b, pl.ds(j, 16)] = dstage[st, pl.ds(j, 16)]
                gidx[b, pl.ds(j, 16)] = sstage[st, pl.ds(j, 16)]

        def gen_edge(b, st, k_tr):
            for j in range(0, _B, 16):
                didx[b, pl.ds(j, 16)] = dstage[st, pl.ds(j, 16)]

        def issue_g(b, k_tr):
            if b % 2 == 0:
                pltpu.async_copy(node_hbm.at[gidx.at[b], pl.ds(colbase, h)],
                                 buf.at[b], gsem[b])
            else:
                pltpu.async_copy(
                    edge_hbm.at[pl.ds(base0 + k_tr * _B, _B), pl.ds(colbase, h)],
                    buf.at[b], gsem[b])

        def wait_g(b):
            if b % 2 == 0:
                pltpu.make_async_copy(node_hbm.at[gidx.at[b], pl.ds(colbase, h)],
                                      buf.at[b], gsem[b]).wait()
            else:
                pltpu.make_async_copy(
                    edge_hbm.at[pl.ds(base0, _B), pl.ds(colbase, h)],
                    buf.at[b], gsem[b]).wait()

        def issue_sc(b):
            pltpu.async_copy(buf.at[b], acc.at[didx.at[b]], ssem[b], add=True)

        def wait_sc(b):
            pltpu.make_async_copy(buf.at[b], acc.at[didx.at[b]], ssem[b]).wait()

        issue_ld(0, 0)
        issue_ld(1, 1)

        @pl.loop(0, _B)
        def _(r):
            for j in range(0, h, 16):
                buf[0, r, pl.ds(j, 16)] = jnp.zeros((16,), jnp.float32)

        nfull = rows_pt // _B
        ztail = rows_pt - nfull * _B
        for m in range(nfull):
            pltpu.async_copy(buf.at[0], acc.at[pl.ds(arow + m * _B, _B)], zsem)
        if ztail:
            pltpu.async_copy(buf.at[0, pl.ds(0, ztail)],
                             acc.at[pl.ds(arow + nfull * _B, ztail)], zsem)
        for m in range(nfull):
            pltpu.make_async_copy(buf.at[0],
                                  acc.at[pl.ds(arow + m * _B, _B)], zsem).wait()
        if ztail:
            pltpu.make_async_copy(buf.at[0, pl.ds(0, ztail)],
                                  acc.at[pl.ds(arow + nfull * _B, ztail)],
                                  zsem).wait()
        wait_ld(0)
        gen_node(0, 0)
        issue_g(0, 0)
        gen_edge(1, 0, 0)
        issue_g(1, 0)
        issue_ld(2, 0)
        wait_ld(1)
        gen_node(2, 1)
        issue_g(2, 1)
        gen_edge(3, 1, 1)
        issue_g(3, 1)
        issue_ld(3, 1)
        plsc.subcore_barrier()
        wait_g(0)
        issue_sc(0)
        wait_g(1)
        issue_sc(1)
        wait_g(2)
        issue_sc(2)

        @pl.loop(4, njobs - 2, step=4)
        def _(g0_tr):
            kbase = g0_tr // 2
            for b in range(4):
                k_tr = kbase + (b // 2)
                st = b // 2
                if b % 2 == 0:
                    wait_sc(b)
                    wait_ld(st)
                    gen_node(b, st)
                    issue_g(b, k_tr)
                else:
                    wait_sc(b)
                    gen_edge(b, st, k_tr)
                    issue_g(b, k_tr)

                    @pl.when(k_tr + 2 < nchunks)
                    def _():
                        issue_ld(k_tr + 2, st)
                bp = (b + 3) % 4
                wait_g(bp)
                issue_sc(bp)

        wait_sc(0)
        wait_ld(0)
        gen_node(0, 0)
        issue_g(0, nchunks - 1)
        wait_g(3)
        issue_sc(3)
        wait_sc(1)
        gen_edge(1, 0, nchunks - 1)
        issue_g(1, nchunks - 1)
        wait_g(0)
        issue_sc(0)
        wait_g(1)
        issue_sc(1)
        wait_sc(2)
        wait_sc(3)
        wait_sc(0)
        wait_sc(1)

        plsc.subcore_barrier()

        def writeout(sizes):
            off = 0
            for i, sz in enumerate(sizes):
                wb = i % 2
                if i >= 2:
                    psz = sizes[i - 2]
                    poff = sum(sizes[:i - 2])
                    pltpu.make_async_copy(
                        buf.at[wb, pl.ds(0, psz)],
                        out_hbm.at[pl.ds(arow + poff, psz), pl.ds(colbase, h)],
                        gsem[wb]).wait()
                pltpu.sync_copy(acc.at[pl.ds(arow + off, sz)],
                                buf.at[wb, pl.ds(0, sz)])
                pltpu.async_copy(
                    buf.at[wb, pl.ds(0, sz)],
                    out_hbm.at[pl.ds(arow + off, sz), pl.ds(colbase, h)],
                    gsem[wb])
                off += sz
            for i in range(max(0, len(sizes) - 2), len(sizes)):
                wb = i % 2
                sz = sizes[i]
                poff = sum(sizes[:i])
                pltpu.make_async_copy(
                    buf.at[wb, pl.ds(0, sz)],
                    out_hbm.at[pl.ds(arow + poff, sz), pl.ds(colbase, h)],
                    gsem[wb]).wait()

        @pl.when(s < _TILES - 1)
        def _():
            writeout([_B] * (rows_pt // _B) + [rows_pt % _B])

        @pl.when(s == _TILES - 1)
        def _():
            writeout([_B] * (rows_last // _B) + [rows_last % _B])

    return k(node_feat, edge_feat, edge_index.reshape(2 * e))
